# baseline (speedup 1.0000x reference)
import functools

import numpy as np

import jax
import jax.numpy as jnp
from jax.experimental import pallas as pl
from jax.experimental.pallas import tpu as pltpu
from jax.sharding import Mesh, PartitionSpec as P

try:
    from jax import shard_map as _shard_map
except ImportError:
    from jax.experimental.shard_map import shard_map as _shard_map

_K = 3
_ITERS = 50
_BAGS_LEN = 1042
_CLUS_LEN = 961
_TGT_LEN = _BAGS_LEN - _CLUS_LEN
_D = 768
_B = 4

_HIGHEST = jax.lax.Precision.HIGHEST


def _tic_mil_kernel(nbags, clus_ref, tgt_ref, u_ref, w_ref, b_ref,
                    out_ref, mind_ref, nonmind_ref):
    f32 = jnp.float32
    bf16 = jnp.bfloat16

    cl = [clus_ref[b] for b in range(nbags)]

    cl_hi, cl_mid, cl_lo = [], [], []
    for b in range(nbags):
        hi = cl[b].astype(bf16)
        r1 = cl[b] - hi.astype(f32)
        mid = r1.astype(bf16)
        lo = (r1 - mid.astype(f32)).astype(bf16)
        cl_hi.append(hi)
        cl_mid.append(mid)
        cl_lo.append(lo)

    centers0 = []
    for b in range(nbags):
        col_max = jnp.max(cl[b], axis=0)
        col_min = jnp.min(cl[b], axis=0)
        centers0.append(col_max[None, :] + u_ref[b] * (col_min[None, :] - col_max[None, :]))

    def _row_sum_xla_assoc(sq):
        acc = sq[:, 0:128]
        for c in range(1, 6):
            acc = acc + sq[:, 128 * c:128 * (c + 1)]
        m = acc
        for j in range(1, 16):
            m = m + pltpu.roll(acc, 128 - 8 * j, axis=1)
        b2 = m + pltpu.roll(m, 124, axis=1)
        c2 = b2 + pltpu.roll(b2, 126, axis=1)
        d2 = c2 + pltpu.roll(c2, 127, axis=1)
        return d2[:, 0]

    def assign_from_centers(b, centers_b):
        dists = []
        for k in range(_K):
            diff = cl[b] - centers_b[k][None, :]
            dists.append(jnp.sqrt(_row_sum_xla_assoc(diff * diff)))
        best = dists[0]
        idx = jnp.zeros((_CLUS_LEN,), dtype=jnp.int32)
        for k in range(1, _K):
            m = dists[k] < best
            idx = jnp.where(m, k, idx)
            best = jnp.where(m, dists[k], best)
        return idx

    def _seg_matmul(oh_bf, b):
        dn = (((1,), (0,)), ((), ()))
        s = jax.lax.dot_general(oh_bf, cl_hi[b], dn, preferred_element_type=f32)
        s = s + jax.lax.dot_general(oh_bf, cl_mid[b], dn, preferred_element_type=f32)
        s = s + jax.lax.dot_general(oh_bf, cl_lo[b], dn, preferred_element_type=f32)
        return s

    def body(state):
        it, _stable, centers, prev = state
        new_assign = []
        new_centers = []
        for b in range(nbags):
            a = assign_from_centers(b, centers[b])
            new_assign.append(a)
            ohm = (jax.lax.broadcasted_iota(jnp.int32, (_K, _CLUS_LEN), 0)
                   == a[None, :])
            sums = _seg_matmul(ohm.astype(bf16), b)
            counts = jnp.sum(ohm.astype(f32), axis=1)
            newc = jnp.where(counts[:, None] > 0,
                             sums / jnp.maximum(counts, 1.0)[:, None],
                             centers[b])
            new_centers.append(newc)
        stable = jnp.bool_(True)
        for b in range(nbags):
            stable = stable & jnp.all(new_assign[b] == prev[b])
        return (it + 1, stable, tuple(new_centers), tuple(new_assign))

    def cond(state):
        it, stable, _c, _a = state
        return (it < _ITERS) & jnp.logical_not(stable)

    init_assign = tuple(jnp.full((_CLUS_LEN,), -1, dtype=jnp.int32)
                        for _ in range(nbags))
    _it, _st, _centers, assign = jax.lax.while_loop(
        cond, body, (jnp.int32(0), jnp.bool_(False), tuple(centers0), init_assign))

    pooled = []
    dmins = []
    dsums = []
    for b in range(nbags):
        tg = tgt_ref[b]
        t_mean = jnp.sum(tg) / f32(_TGT_LEN * _D)
        rs = jnp.sum(cl[b], axis=1)
        dis = []
        for k in range(_K):
            mask = (assign[b] == k).astype(f32)
            cnt = jnp.sum(mask)
            csum = jnp.sum(mask * rs)
            cmean = jnp.where(cnt > 0,
                              csum / jnp.maximum(cnt * f32(_D), 1.0),
                              f32(0.0))
            dis.append(jnp.abs(t_mean - cmean))
        dmins.append(jnp.minimum(jnp.minimum(dis[0], dis[1]), dis[2]))
        dsums.append(dis[0] + dis[1] + dis[2])
        scale = jnp.zeros((_CLUS_LEN,), dtype=f32)
        for k in range(_K):
            scale = scale + (assign[b] == k).astype(f32) * (f32(1.0) - dis[k])
        clus_scaled = cl[b] * scale[:, None]
        pooled.append(((jnp.sum(clus_scaled, axis=0) + jnp.sum(tg, axis=0))
                       / f32(_BAGS_LEN))[None, :])

    feat = jnp.concatenate(pooled, axis=0) if nbags > 1 else pooled[0]
    out = jax.lax.dot_general(feat, w_ref[...], (((1,), (1,)), ((), ())),
                              precision=_HIGHEST,
                              preferred_element_type=f32)
    out_ref[...] = out + b_ref[...]

    msum = dmins[0]
    ssum = dsums[0]
    for b in range(1, nbags):
        msum = msum + dmins[b]
        ssum = ssum + dsums[b]
    mind_ref[...] = jnp.reshape(msum, (1, 1))
    nonmind_ref[...] = jnp.reshape(ssum - msum, (1, 1))


def _run_bags(clus, tgt, u, head_W, head_b2, interpret):
    nbags = clus.shape[0]
    return pl.pallas_call(
        functools.partial(_tic_mil_kernel, nbags),
        out_shape=(
            jax.ShapeDtypeStruct((nbags, _K), jnp.float32),
            jax.ShapeDtypeStruct((1, 1), jnp.float32),
            jax.ShapeDtypeStruct((1, 1), jnp.float32),
        ),
        interpret=interpret,
    )(clus, tgt, u, head_W, head_b2)


@functools.partial(jax.jit, static_argnames=("interpret",))
def kernel(x, head_W, head_b, interpret=False):
    B = x.shape[0] // _BAGS_LEN
    y = jnp.reshape(x, (B, _BAGS_LEN, _D))
    clus = y[:, :_CLUS_LEN, :]
    tgt = y[:, _CLUS_LEN:, :]
    u = jnp.stack([
        jax.random.uniform(jax.random.fold_in(jax.random.key(42), i),
                           (_K, _D), dtype=jnp.float32)
        for i in range(B)])
    head_b2 = jnp.reshape(head_b, (1, _K))

    devs = jax.devices()
    n_shard = 2 if (len(devs) >= 2 and B % 2 == 0 and not interpret) else 1
    if n_shard == 2:
        mesh = Mesh(np.array(devs[:2]), ("d",))
        fn = _shard_map(
            functools.partial(_run_bags, interpret=interpret),
            mesh=mesh,
            in_specs=(P("d"), P("d"), P("d"), P(), P()),
            out_specs=(P("d"), P("d"), P("d")),
            check_vma=False,
        )
        out, mind, nonmind = fn(clus, tgt, u, head_W, head_b2)
        min_dis = jnp.reshape((mind[0, 0] + mind[1, 0]) / jnp.float32(B), (1,))
        non_min_dis = jnp.reshape((nonmind[0, 0] + nonmind[1, 0])
                                  / jnp.float32(B), (1,))
    else:
        out, mind, nonmind = _run_bags(clus, tgt, u, head_W, head_b2, interpret)
        min_dis = jnp.reshape(mind[0, 0] / jnp.float32(B), (1,))
        non_min_dis = jnp.reshape(nonmind[0, 0] / jnp.float32(B), (1,))
    return (out, min_dis, non_min_dis)


# speedup vs baseline: 1.2881x; 1.2881x over previous
import functools

import numpy as np

import jax
import jax.numpy as jnp
from jax.experimental import pallas as pl
from jax.experimental.pallas import tpu as pltpu
from jax.sharding import Mesh, PartitionSpec as P

try:
    from jax import shard_map as _shard_map
except ImportError:
    from jax.experimental.shard_map import shard_map as _shard_map

_K = 3
_ITERS = 50
_BAGS_LEN = 1042
_CLUS_LEN = 961
_TGT_LEN = _BAGS_LEN - _CLUS_LEN
_D = 768
_B = 4

_HIGHEST = jax.lax.Precision.HIGHEST


def _tic_mil_kernel(nbags, clt_ref, tgt_ref, u_ref, w_ref, b_ref,
                    out_ref, mind_ref, nonmind_ref):
    f32 = jnp.float32
    bf16 = jnp.bfloat16

    cl = [clt_ref[b] for b in range(nbags)]

    cl_hi, cl_mid, cl_lo = [], [], []
    for b in range(nbags):
        hi = cl[b].astype(bf16)
        r1 = cl[b] - hi.astype(f32)
        mid = r1.astype(bf16)
        lo = (r1 - mid.astype(f32)).astype(bf16)
        cl_hi.append(hi)
        cl_mid.append(mid)
        cl_lo.append(lo)

    centers0 = []
    for b in range(nbags):
        col_max = jnp.max(cl[b], axis=1, keepdims=True)
        col_min = jnp.min(cl[b], axis=1, keepdims=True)
        centers0.append(col_max + u_ref[b] * (col_min - col_max))

    def _row_sum_xla_assoc(sq):
        acc = sq[0:128, :]
        for c in range(1, 6):
            acc = acc + sq[128 * c:128 * (c + 1), :]
        m = acc[0:8, :]
        for j in range(1, 16):
            m = m + acc[8 * j:8 * (j + 1), :]
        b2 = m + pltpu.roll(m, 4, axis=0)
        c2 = b2 + pltpu.roll(b2, 6, axis=0)
        d2 = c2 + pltpu.roll(c2, 7, axis=0)
        return d2[0:1, :]

    def assign_from_centers(b, centers_b):
        dists = []
        for k in range(_K):
            diff = cl[b] - centers_b[:, k:k + 1]
            dists.append(jnp.sqrt(_row_sum_xla_assoc(diff * diff)))
        best = dists[0]
        idx = jnp.zeros((1, _CLUS_LEN), dtype=jnp.int32)
        for k in range(1, _K):
            mk = dists[k] < best
            idx = jnp.where(mk, k, idx)
            best = jnp.where(mk, dists[k], best)
        return idx

    def _seg_matmul(oht, b):
        dn = (((1,), (0,)), ((), ()))
        s = jax.lax.dot_general(cl_hi[b], oht, dn, preferred_element_type=f32)
        s = s + jax.lax.dot_general(cl_mid[b], oht, dn, preferred_element_type=f32)
        s = s + jax.lax.dot_general(cl_lo[b], oht, dn, preferred_element_type=f32)
        return s

    def body(state):
        it, _stable, centers, prev = state
        new_assign = []
        new_centers = []
        for b in range(nbags):
            a = assign_from_centers(b, centers[b])
            new_assign.append(a)
            a_col = jnp.transpose(a)
            ohm = (jax.lax.broadcasted_iota(jnp.int32, (_CLUS_LEN, _K), 1)
                   == a_col)
            sums = _seg_matmul(ohm.astype(bf16), b)
            counts = jnp.sum(ohm.astype(f32), axis=0, keepdims=True)
            newc = jnp.where(counts > 0,
                             sums / jnp.maximum(counts, 1.0),
                             centers[b])
            new_centers.append(newc)
        stable = jnp.bool_(True)
        for b in range(nbags):
            stable = stable & jnp.all(new_assign[b] == prev[b])
        return (it + 1, stable, tuple(new_centers), tuple(new_assign))

    def cond(state):
        it, stable, _c, _a = state
        return (it < _ITERS) & jnp.logical_not(stable)

    init_assign = tuple(jnp.full((1, _CLUS_LEN), -1, dtype=jnp.int32)
                        for _ in range(nbags))
    _it, _st, _centers, assign = jax.lax.while_loop(
        cond, body, (jnp.int32(0), jnp.bool_(False), tuple(centers0), init_assign))

    pooled = []
    dmins = []
    dsums = []
    for b in range(nbags):
        tg = tgt_ref[b]
        t_mean = jnp.sum(tg) / f32(_TGT_LEN * _D)
        rs = jnp.sum(cl[b], axis=0, keepdims=True)
        dis = []
        for k in range(_K):
            mask = (assign[b] == k).astype(f32)
            cnt = jnp.sum(mask)
            csum = jnp.sum(mask * rs)
            cmean = jnp.where(cnt > 0,
                              csum / jnp.maximum(cnt * f32(_D), 1.0),
                              f32(0.0))
            dis.append(jnp.abs(t_mean - cmean))
        dmins.append(jnp.minimum(jnp.minimum(dis[0], dis[1]), dis[2]))
        dsums.append(dis[0] + dis[1] + dis[2])
        scale = jnp.zeros((1, _CLUS_LEN), dtype=f32)
        for k in range(_K):
            scale = scale + (assign[b] == k).astype(f32) * (f32(1.0) - dis[k])
        scaled_sum = jnp.sum(cl[b] * scale, axis=1, keepdims=True)
        pooled.append((scaled_sum + jnp.sum(tg, axis=1, keepdims=True))
                      / f32(_BAGS_LEN))

    feat = jnp.concatenate(pooled, axis=1)
    outt = jax.lax.dot_general(w_ref[...], feat, (((1,), (0,)), ((), ())),
                               precision=_HIGHEST,
                               preferred_element_type=f32)
    out_ref[...] = outt + b_ref[...]

    msum = dmins[0]
    ssum = dsums[0]
    for b in range(1, nbags):
        msum = msum + dmins[b]
        ssum = ssum + dsums[b]
    mind_ref[...] = jnp.reshape(msum, (1, 1))
    nonmind_ref[...] = jnp.reshape(ssum - msum, (1, 1))


def _run_bags(clt, tgt, u, head_W, head_b2, interpret):
    nbags = clt.shape[0]
    return pl.pallas_call(
        functools.partial(_tic_mil_kernel, nbags),
        out_shape=(
            jax.ShapeDtypeStruct((_K, nbags), jnp.float32),
            jax.ShapeDtypeStruct((1, 1), jnp.float32),
            jax.ShapeDtypeStruct((1, 1), jnp.float32),
        ),
        interpret=interpret,
    )(clt, tgt, u, head_W, head_b2)


@functools.partial(jax.jit, static_argnames=("interpret",))
def kernel(x, head_W, head_b, interpret=False):
    B = x.shape[0] // _BAGS_LEN
    y = jnp.reshape(x, (B, _BAGS_LEN, _D))
    clt = jnp.transpose(y[:, :_CLUS_LEN, :], (0, 2, 1))
    tgt = jnp.transpose(y[:, _CLUS_LEN:, :], (0, 2, 1))
    u = jnp.stack([
        jnp.transpose(jax.random.uniform(
            jax.random.fold_in(jax.random.key(42), i),
            (_K, _D), dtype=jnp.float32))
        for i in range(B)])
    head_b2 = jnp.reshape(head_b, (_K, 1))

    devs = jax.devices()
    n_shard = 2 if (len(devs) >= 2 and B % 2 == 0 and not interpret) else 1
    if n_shard == 2:
        mesh = Mesh(np.array(devs[:2]), ("d",))
        fn = _shard_map(
            functools.partial(_run_bags, interpret=interpret),
            mesh=mesh,
            in_specs=(P("d"), P("d"), P("d"), P(), P()),
            out_specs=(P(None, "d"), P("d"), P("d")),
            check_vma=False,
        )
        outt, mind, nonmind = fn(clt, tgt, u, head_W, head_b2)
        min_dis = jnp.reshape((mind[0, 0] + mind[1, 0]) / jnp.float32(B), (1,))
        non_min_dis = jnp.reshape((nonmind[0, 0] + nonmind[1, 0])
                                  / jnp.float32(B), (1,))
    else:
        outt, mind, nonmind = _run_bags(clt, tgt, u, head_W, head_b2, interpret)
        min_dis = jnp.reshape(mind[0, 0] / jnp.float32(B), (1,))
        non_min_dis = jnp.reshape(nonmind[0, 0] / jnp.float32(B), (1,))
    out = jnp.transpose(outt)
    return (out, min_dis, non_min_dis)


# speedup vs baseline: 3.9993x; 3.1048x over previous
import functools

import numpy as np

import jax
import jax.numpy as jnp
from jax.experimental import pallas as pl
from jax.experimental.pallas import tpu as pltpu
from jax.sharding import Mesh, PartitionSpec as P

try:
    from jax import shard_map as _shard_map
except ImportError:
    from jax.experimental.shard_map import shard_map as _shard_map

_K = 3
_ITERS = 50
_BAGS_LEN = 1042
_CLUS_LEN = 961
_TGT_LEN = _BAGS_LEN - _CLUS_LEN
_D = 768
_B = 4

_HIGHEST = jax.lax.Precision.HIGHEST


def _tic_mil_kernel(nbags, clt_ref, tgt_ref, u_ref, w_ref, b_ref,
                    out_ref, mind_ref, nonmind_ref):
    f32 = jnp.float32
    bf16 = jnp.bfloat16

    cl = [clt_ref[b] for b in range(nbags)]

    cl_hi, cl_mid, cl_lo = [], [], []
    for b in range(nbags):
        hi = cl[b].astype(bf16)
        r1 = cl[b] - hi.astype(f32)
        mid = r1.astype(bf16)
        lo = (r1 - mid.astype(f32)).astype(bf16)
        cl_hi.append(hi)
        cl_mid.append(mid)
        cl_lo.append(lo)

    centers0 = []
    for b in range(nbags):
        col_max = jnp.max(cl[b], axis=1, keepdims=True)
        col_min = jnp.min(cl[b], axis=1, keepdims=True)
        centers0.append(col_max + u_ref[b] * (col_min - col_max))

    def _row_sum_xla_assoc(sq):
        acc = sq[0:128, :]
        for c in range(1, 6):
            acc = acc + sq[128 * c:128 * (c + 1), :]
        m = acc[0:8, :]
        for j in range(1, 16):
            m = m + acc[8 * j:8 * (j + 1), :]
        b2 = m + pltpu.roll(m, 4, axis=0)
        c2 = b2 + pltpu.roll(b2, 6, axis=0)
        d2 = c2 + pltpu.roll(c2, 7, axis=0)
        return d2[0:1, :]

    def assign_from_centers(b, centers_b):
        dists = []
        for k in range(_K):
            diff = cl[b] - centers_b[:, k:k + 1]
            dists.append(jnp.sqrt(_row_sum_xla_assoc(diff * diff)))
        best = dists[0]
        idx = jnp.zeros((1, _CLUS_LEN), dtype=jnp.int32)
        for k in range(1, _K):
            mk = dists[k] < best
            idx = jnp.where(mk, k, idx)
            best = jnp.where(mk, dists[k], best)
        return idx

    def _seg_matmul(oht, b):
        dn = (((1,), (0,)), ((), ()))
        s = jax.lax.dot_general(cl_hi[b], oht, dn, preferred_element_type=f32)
        s = s + jax.lax.dot_general(cl_mid[b], oht, dn, preferred_element_type=f32)
        s = s + jax.lax.dot_general(cl_lo[b], oht, dn, preferred_element_type=f32)
        return s

    def body(state):
        it, _stable, centers, prev = state
        new_assign = []
        new_centers = []
        for b in range(nbags):
            a = assign_from_centers(b, centers[b])
            new_assign.append(a)
            a_col = jnp.transpose(a)
            ohm = (jax.lax.broadcasted_iota(jnp.int32, (_CLUS_LEN, _K), 1)
                   == a_col)
            sums = _seg_matmul(ohm.astype(bf16), b)
            counts = jnp.sum(ohm.astype(f32), axis=0, keepdims=True)
            newc = jnp.where(counts > 0,
                             sums / jnp.maximum(counts, 1.0),
                             centers[b])
            new_centers.append(newc)
        stable = jnp.bool_(True)
        for b in range(nbags):
            stable = stable & jnp.all(new_assign[b] == prev[b])
        return (it + 1, stable, tuple(new_centers), tuple(new_assign))

    def cond(state):
        it, stable, _c, _a = state
        return (it < _ITERS) & jnp.logical_not(stable)

    init_assign = tuple(jnp.full((1, _CLUS_LEN), -1, dtype=jnp.int32)
                        for _ in range(nbags))
    _it, _st, _centers, assign = jax.lax.while_loop(
        cond, body, (jnp.int32(0), jnp.bool_(False), tuple(centers0), init_assign))

    pooled = []
    dmins = []
    dsums = []
    for b in range(nbags):
        tg = tgt_ref[b]
        t_mean = jnp.sum(tg) / f32(_TGT_LEN * _D)
        rs = jnp.sum(cl[b], axis=0, keepdims=True)
        dis = []
        for k in range(_K):
            mask = (assign[b] == k).astype(f32)
            cnt = jnp.sum(mask)
            csum = jnp.sum(mask * rs)
            cmean = jnp.where(cnt > 0,
                              csum / jnp.maximum(cnt * f32(_D), 1.0),
                              f32(0.0))
            dis.append(jnp.abs(t_mean - cmean))
        dmins.append(jnp.minimum(jnp.minimum(dis[0], dis[1]), dis[2]))
        dsums.append(dis[0] + dis[1] + dis[2])
        scale = jnp.zeros((1, _CLUS_LEN), dtype=f32)
        for k in range(_K):
            scale = scale + (assign[b] == k).astype(f32) * (f32(1.0) - dis[k])
        scaled_sum = jnp.sum(cl[b] * scale, axis=1, keepdims=True)
        pooled.append((scaled_sum + jnp.sum(tg, axis=1, keepdims=True))
                      / f32(_BAGS_LEN))

    feat = jnp.concatenate(pooled, axis=1)
    outt = jax.lax.dot_general(w_ref[...], feat, (((1,), (0,)), ((), ())),
                               precision=_HIGHEST,
                               preferred_element_type=f32)
    out_ref[...] = outt + b_ref[...]

    msum = dmins[0]
    ssum = dsums[0]
    for b in range(1, nbags):
        msum = msum + dmins[b]
        ssum = ssum + dsums[b]
    mind_ref[...] = jnp.reshape(msum, (1, 1))
    nonmind_ref[...] = jnp.reshape(ssum - msum, (1, 1))


def _run_bags(clt, tgt, u, head_W, head_b2, interpret):
    nbags = clt.shape[0]
    return pl.pallas_call(
        functools.partial(_tic_mil_kernel, nbags),
        out_shape=(
            jax.ShapeDtypeStruct((_K, nbags), jnp.float32),
            jax.ShapeDtypeStruct((1, 1), jnp.float32),
            jax.ShapeDtypeStruct((1, 1), jnp.float32),
        ),
        interpret=interpret,
    )(clt, tgt, u, head_W, head_b2)


@functools.partial(jax.jit, static_argnames=("interpret",))
def kernel(x, head_W, head_b, interpret=False):
    B = x.shape[0] // _BAGS_LEN
    y = jnp.reshape(x, (B, _BAGS_LEN, _D))
    clt = jnp.transpose(y[:, :_CLUS_LEN, :], (0, 2, 1))
    tgt = jnp.transpose(y[:, _CLUS_LEN:, :], (0, 2, 1))
    u = jnp.stack([
        jnp.transpose(jax.random.uniform(
            jax.random.fold_in(jax.random.key(42), i),
            (_K, _D), dtype=jnp.float32))
        for i in range(B)])
    head_b2 = jnp.reshape(head_b, (_K, 1))

    devs = jax.devices()
    n_shard = 1 if (len(devs) >= 2 and B % 2 == 0 and not interpret) else 1
    if n_shard == 2:
        mesh = Mesh(np.array(devs[:2]), ("d",))
        fn = _shard_map(
            functools.partial(_run_bags, interpret=interpret),
            mesh=mesh,
            in_specs=(P("d"), P("d"), P("d"), P(), P()),
            out_specs=(P(None, "d"), P("d"), P("d")),
            check_vma=False,
        )
        outt, mind, nonmind = fn(clt, tgt, u, head_W, head_b2)
        min_dis = jnp.reshape((mind[0, 0] + mind[1, 0]) / jnp.float32(B), (1,))
        non_min_dis = jnp.reshape((nonmind[0, 0] + nonmind[1, 0])
                                  / jnp.float32(B), (1,))
    else:
        outt, mind, nonmind = _run_bags(clt, tgt, u, head_W, head_b2, interpret)
        min_dis = jnp.reshape(mind[0, 0] / jnp.float32(B), (1,))
        non_min_dis = jnp.reshape(nonmind[0, 0] / jnp.float32(B), (1,))
    out = jnp.transpose(outt)
    return (out, min_dis, non_min_dis)
